# baseline (speedup 1.0000x reference)
import jax
import jax.numpy as jnp
from jax import lax
from jax.experimental import pallas as pl
from jax.experimental.pallas import tpu as pltpu
from jax.experimental.pallas import tpu_sc as plsc

BATCH_NUM = 1024
WIN_SIZE = 50
EMBED_DIM = 64
L = 16

def _body(ids_hbm, bat_hbm, win_hbm, table_hbm, out_hbm, bout_v, sem):
  s = lax.axis_index("s")
  zero16 = jnp.zeros((L,), jnp.float32)
  bout_v[pl.ds(0, L)] = zero16
  pltpu.async_copy(bout_v, out_hbm.at[0, pl.ds(0, L)], sem).wait()

_mesh = plsc.VectorSubcoreMesh(core_axis_name="c", subcore_axis_name="s",
                               num_cores=2, num_subcores=16)

_pooling = pl.kernel(
    _body,
    out_type=jax.ShapeDtypeStruct((BATCH_NUM, EMBED_DIM * WIN_SIZE),
                                  jnp.float32),
    mesh=_mesh,
    compiler_params=pltpu.CompilerParams(use_tc_tiling_on_sc=False,
                                         needs_layout_passes=False),
    scratch_types=[
        pltpu.VMEM((L,), jnp.float32),
        pltpu.SemaphoreType.DMA,
    ],
)

@jax.jit
def kernel(input, batch_i, win_i, table):
  tlin = table.reshape(-1).reshape(1000000, EMBED_DIM)
  out = _pooling(input, batch_i, win_i, tlin)
  return out.reshape(BATCH_NUM, EMBED_DIM, WIN_SIZE)
